# baseline (speedup 1.0000x reference)
import functools

import numpy as np
import jax
import jax.numpy as jnp
from jax import lax
from jax.experimental import pallas as pl
from jax.experimental.pallas import tpu as pltpu
from jax.experimental.pallas import tpu_sc as plsc

_VOCAB = 100000
_D = 128
_MAXLEN = 2048
_N_PARAM = 10000
_BATCH = 4
_SEQ = 2048
_EPS = 1e-12

_L = 16
_NV = _D // _L
_NC = 2
_NW = _NC * 16
_ROWS = _BATCH * _SEQ
_RPW = _ROWS // _NW
_GCH = 128
_NCH = _RPW // _GCH
_UNR = 2


def _make_pe_np():
    k = np.arange(_MAXLEN, dtype=np.float32)[:, None]
    div = np.exp(
        np.arange(0, _D, 2, dtype=np.float32) * (-np.log(_N_PARAM) / _D)
    )
    pe = np.zeros((_MAXLEN, _D), dtype=np.float32)
    pe[:, 0::2] = np.sin(k * div)
    pe[:, 1::2] = np.cos(k * div)
    return pe


_PE = _make_pe_np()

_GDN = lax.GatherDimensionNumbers(
    offset_dims=(), collapsed_slice_dims=(0,), start_index_map=(0,)
)


def _make_bfly():
    lane = lax.iota(jnp.int32, _L)
    return [jnp.reshape(lane ^ k, (_L, 1)) for k in (1, 2, 4, 8)]


def _allsum(v, bfly):
    for perm in bfly:
        v = v + lax.gather(
            v, perm, _GDN, (1,),
            mode=lax.GatherScatterMode.PROMISE_IN_BOUNDS,
        )
    return v


def _sc_body(ids_hbm, table_hbm, pe_hbm, gamma_hbm, beta_hbm, out_hbm,
             idx_v, rows_v, pe_v, gamma_v, beta_v, g0, g1, psem, wsem):
    gsems = [g0, g1]
    c = lax.axis_index("c")
    s = lax.axis_index("s")
    wid = s * _NC + c
    base = wid * _RPW
    b = base // _SEQ
    col = base % _SEQ

    for j in range(_NCH):
        pltpu.sync_copy(
            ids_hbm.at[b, pl.ds(col + j * _GCH, _GCH)], idx_v.at[j]
        )

    gathers = [
        pltpu.async_copy(
            table_hbm.at[idx_v.at[j]],
            rows_v.at[pl.ds(j * _GCH, _GCH)],
            gsems[j],
        )
        for j in range(_NCH)
    ]
    pe_cp = pltpu.async_copy(pe_hbm.at[pl.ds(col, _RPW)], pe_v, psem)
    pltpu.sync_copy(gamma_hbm, gamma_v)
    pltpu.sync_copy(beta_hbm, beta_v)

    gammas = [gamma_v[pl.ds(j * _L, _L)] for j in range(_NV)]
    betas = [beta_v[pl.ds(j * _L, _L)] for j in range(_NV)]
    bfly = _make_bfly()
    inv_d = jnp.float32(1.0 / _D)

    def do_row(r):
        xs = [
            rows_v[r, pl.ds(j * _L, _L)] + pe_v[r, pl.ds(j * _L, _L)]
            for j in range(_NV)
        ]
        sv = (xs[0] + xs[1]) + (xs[2] + xs[3])
        sw = (xs[4] + xs[5]) + (xs[6] + xs[7])
        sq = [x * x for x in xs]
        qv = (sq[0] + sq[1]) + (sq[2] + sq[3])
        qw = (sq[4] + sq[5]) + (sq[6] + sq[7])
        mean_v = _allsum(sv + sw, bfly) * inv_d
        var_v = _allsum(qv + qw, bfly) * inv_d - mean_v * mean_v
        xv = var_v + jnp.float32(_EPS)
        iv = plsc.bitcast(xv, jnp.int32)
        iv = jnp.int32(0x5F3759DF) - lax.shift_right_logical(iv, 1)
        y = plsc.bitcast(iv, jnp.float32)
        half_x = xv * jnp.float32(0.5)
        for _i in range(3):
            y = y * (jnp.float32(1.5) - half_x * y * y)
        for j in range(_NV):
            rows_v[r, pl.ds(j * _L, _L)] = (
                (xs[j] - mean_v) * y * gammas[j] + betas[j]
            )

    pe_cp.wait()
    writes = []
    for ch in range(_NCH):
        gathers[ch].wait()

        def chunk_body(i, carry, _ch=ch):
            r0 = _ch * _GCH + i * _UNR
            for rr in range(_UNR):
                do_row(r0 + rr)
            return carry

        lax.fori_loop(0, _GCH // _UNR, chunk_body, 0)
        writes.append(
            pltpu.async_copy(
                rows_v.at[pl.ds(ch * _GCH, _GCH)],
                out_hbm.at[pl.ds(base + ch * _GCH, _GCH)],
                wsem,
            )
        )
    for w in writes:
        w.wait()


@jax.jit
def _embed_ln(ids, table, pe, gamma, beta):
    mesh = plsc.VectorSubcoreMesh(
        core_axis_name="c", subcore_axis_name="s", num_cores=_NC
    )
    return pl.kernel(
        _sc_body,
        out_type=jax.ShapeDtypeStruct((_ROWS, _D), jnp.float32),
        mesh=mesh,
        scratch_types=[
            pltpu.VMEM((_NCH, _GCH), jnp.int32),
            pltpu.VMEM((_RPW, _D), jnp.float32),
            pltpu.VMEM((_RPW, _D), jnp.float32),
            pltpu.VMEM((_D,), jnp.float32),
            pltpu.VMEM((_D,), jnp.float32),
            pltpu.SemaphoreType.DMA,
            pltpu.SemaphoreType.DMA,
            pltpu.SemaphoreType.DMA,
            pltpu.SemaphoreType.DMA,
        ],
        compiler_params=pltpu.CompilerParams(needs_layout_passes=False),
    )(ids, table, pe, gamma, beta)


def kernel(input_ids, table, gamma, beta):
    pe = jnp.asarray(_PE)
    out = _embed_ln(input_ids, table, pe, gamma, beta)
    return out.reshape(_BATCH, _SEQ, _D)


# speedup vs baseline: 1.1118x; 1.1118x over previous
import functools

import numpy as np
import jax
import jax.numpy as jnp
from jax import lax
from jax.experimental import pallas as pl
from jax.experimental.pallas import tpu as pltpu
from jax.experimental.pallas import tpu_sc as plsc

_VOCAB = 100000
_D = 128
_MAXLEN = 2048
_N_PARAM = 10000
_BATCH = 4
_SEQ = 2048
_EPS = 1e-12

_NC = 2
_NW = _NC * 16
_ROWS = _BATCH * _SEQ
_RPW = _ROWS // _NW
_GCH = 128
_NCH = _RPW // _GCH
_TCB = 2048


def _make_pe_np():
    k = np.arange(_MAXLEN, dtype=np.float32)[:, None]
    div = np.exp(
        np.arange(0, _D, 2, dtype=np.float32) * (-np.log(_N_PARAM) / _D)
    )
    pe = np.zeros((_MAXLEN, _D), dtype=np.float32)
    pe[:, 0::2] = np.sin(k * div)
    pe[:, 1::2] = np.cos(k * div)
    return pe


_PE = _make_pe_np()


def _sc_gather_body(ids_hbm, table_hbm, out_hbm, idx_v, rows_v,
                    g0, g1, wsem):
    gsems = [g0, g1]
    c = lax.axis_index("c")
    s = lax.axis_index("s")
    wid = s * _NC + c
    base = wid * _RPW
    b = base // _SEQ
    col = base % _SEQ

    for j in range(_NCH):
        pltpu.sync_copy(
            ids_hbm.at[b, pl.ds(col + j * _GCH, _GCH)], idx_v.at[j]
        )

    gathers = []
    for j in range(_NCH):
        gathers.append(
            pltpu.async_copy(
                table_hbm.at[idx_v.at[j]],
                rows_v.at[pl.ds(j * _GCH, _GCH)],
                gsems[j],
            )
        )
    writes = []
    for j in range(_NCH):
        gathers[j].wait()
        writes.append(
            pltpu.async_copy(
                rows_v.at[pl.ds(j * _GCH, _GCH)],
                out_hbm.at[pl.ds(base + j * _GCH, _GCH)],
                wsem,
            )
        )
    for w in writes:
        w.wait()


def _tc_ln_body(x_ref, pe_ref, g_ref, b_ref, o_ref):
    x = x_ref[...] + pe_ref[...]
    m = jnp.mean(x, axis=-1, keepdims=True)
    v = jnp.mean(x * x, axis=-1, keepdims=True) - m * m
    y = (x - m) * lax.rsqrt(v + jnp.float32(_EPS))
    o_ref[...] = y * g_ref[...] + b_ref[...]


@jax.jit
def _embed_ln(ids, table, pe, gamma, beta):
    mesh = plsc.VectorSubcoreMesh(
        core_axis_name="c", subcore_axis_name="s", num_cores=_NC
    )
    gathered = pl.kernel(
        _sc_gather_body,
        out_type=jax.ShapeDtypeStruct((_ROWS, _D), jnp.float32),
        mesh=mesh,
        scratch_types=[
            pltpu.VMEM((_NCH, _GCH), jnp.int32),
            pltpu.VMEM((_RPW, _D), jnp.float32),
            pltpu.SemaphoreType.DMA,
            pltpu.SemaphoreType.DMA,
            pltpu.SemaphoreType.DMA,
        ],
        compiler_params=pltpu.CompilerParams(needs_layout_passes=False),
    )(ids, table)

    return pl.pallas_call(
        _tc_ln_body,
        grid=(_ROWS // _TCB,),
        in_specs=[
            pl.BlockSpec((_TCB, _D), lambda i: (i, 0)),
            pl.BlockSpec((_SEQ, _D), lambda i: (0, 0)),
            pl.BlockSpec((1, _D), lambda i: (0, 0)),
            pl.BlockSpec((1, _D), lambda i: (0, 0)),
        ],
        out_specs=pl.BlockSpec((_TCB, _D), lambda i: (i, 0)),
        out_shape=jax.ShapeDtypeStruct((_ROWS, _D), jnp.float32),
    )(gathered, pe, gamma.reshape(1, _D), beta.reshape(1, _D))


def kernel(input_ids, table, gamma, beta):
    pe = jnp.asarray(_PE)
    out = _embed_ln(input_ids, table, pe, gamma, beta)
    return out.reshape(_BATCH, _SEQ, _D)
